# baseline (speedup 1.0000x reference)
import functools

import jax
import jax.numpy as jnp
from jax import lax
from jax.experimental import pallas as pl
from jax.experimental.pallas import tpu as pltpu
from jax.experimental.pallas import tpu_sc as plsc

TYPE_VOCAB_SIZE = 2
HIDDEN = 1024
ROWS = 4 * 4096
NUM_WORKERS = 32
ROWS_PER_WORKER = ROWS // NUM_WORKERS
CHUNK = 32
NUM_CHUNKS = ROWS_PER_WORKER // CHUNK


def _make_kernel():
    mesh = plsc.VectorSubcoreMesh(core_axis_name="c", subcore_axis_name="s")

    @functools.partial(
        pl.kernel,
        mesh=mesh,
        out_type=jax.ShapeDtypeStruct((ROWS, HIDDEN), jnp.float32),
        scratch_types=[
            pltpu.VMEM((NUM_CHUNKS, CHUNK), jnp.int32),
            pltpu.VMEM((CHUNK, HIDDEN), jnp.float32),
            pltpu.VMEM((CHUNK, HIDDEN), jnp.float32),
            pltpu.SemaphoreType.DMA,
            pltpu.SemaphoreType.DMA,
            pltpu.SemaphoreType.DMA,
            pltpu.SemaphoreType.DMA,
        ],
    )
    def body(ids_hbm, table_hbm, out_hbm, idx_v, rows_a, rows_b, g_sem_a,
             g_sem_b, s_sem_a, s_sem_b):
        wid = lax.axis_index("s") * 2 + lax.axis_index("c")
        base = wid * ROWS_PER_WORKER
        pltpu.sync_copy(ids_hbm.at[wid], idx_v)
        bufs = (rows_a, rows_b)
        g_sems = (g_sem_a, g_sem_b)
        s_sems = (s_sem_a, s_sem_b)
        gathers = [None] * NUM_CHUNKS
        scatters = [None] * NUM_CHUNKS
        gathers[0] = pltpu.async_copy(table_hbm.at[idx_v.at[0]], bufs[0],
                                      g_sems[0])
        for c in range(NUM_CHUNKS):
            p = c & 1
            if c + 1 < NUM_CHUNKS:
                if c >= 1:
                    scatters[c - 1].wait()
                gathers[c + 1] = pltpu.async_copy(
                    table_hbm.at[idx_v.at[c + 1]], bufs[1 - p], g_sems[1 - p])
            gathers[c].wait()
            scatters[c] = pltpu.async_copy(
                bufs[p], out_hbm.at[pl.ds(base + c * CHUNK, CHUNK)], s_sems[p])
        scatters[NUM_CHUNKS - 2].wait()
        scatters[NUM_CHUNKS - 1].wait()

    return body


_kernel = _make_kernel()


@jax.jit
def kernel(token_type_ids, table):
    b, s = token_type_ids.shape
    ids = token_type_ids.astype(jnp.int32).reshape(NUM_WORKERS, NUM_CHUNKS, CHUNK)
    out = _kernel(ids, table)
    return out.reshape(b, s, HIDDEN)


# speedup vs baseline: 9.5861x; 9.5861x over previous
import functools

import jax
import jax.numpy as jnp
from jax import lax
from jax.experimental import pallas as pl
from jax.experimental.pallas import tpu as pltpu
from jax.experimental.pallas import tpu_sc as plsc

TYPE_VOCAB_SIZE = 2
HIDDEN = 1024
ROWS = 4 * 4096
NUM_WORKERS = 32
ROWS_PER_WORKER = ROWS // NUM_WORKERS


def _make_kernel():
    mesh = plsc.VectorSubcoreMesh(core_axis_name="c", subcore_axis_name="s")

    @functools.partial(
        pl.kernel,
        mesh=mesh,
        out_type=jax.ShapeDtypeStruct((ROWS, HIDDEN), jnp.float32),
        scratch_types=[
            pltpu.VMEM((ROWS_PER_WORKER,), jnp.int32),
            pltpu.VMEM((TYPE_VOCAB_SIZE, HIDDEN), jnp.float32),
            pltpu.SemaphoreType.DMA,
        ],
    )
    def body(ids_hbm, table_hbm, out_hbm, ids_v, table_v, sem):
        wid = lax.axis_index("s") * 2 + lax.axis_index("c")
        base = wid * ROWS_PER_WORKER
        pltpu.sync_copy(ids_hbm.at[wid], ids_v)
        pltpu.sync_copy(table_hbm, table_v)

        def group(g, carry):
            r0 = g * 16
            vec = ids_v[pl.ds(r0, 16)]
            for j in range(16):
                pltpu.async_copy(table_v.at[vec[j]],
                                 out_hbm.at[base + r0 + j], sem)
            return carry

        lax.fori_loop(0, ROWS_PER_WORKER // 16, group, 0)
        pltpu.make_async_copy(
            out_hbm.at[pl.ds(base, ROWS_PER_WORKER)],
            out_hbm.at[pl.ds(base, ROWS_PER_WORKER)],
            sem,
        ).wait()

    return body


_kernel = _make_kernel()


@jax.jit
def kernel(token_type_ids, table):
    b, s = token_type_ids.shape
    ids = token_type_ids.astype(jnp.int32).reshape(NUM_WORKERS, ROWS_PER_WORKER)
    out = _kernel(ids, table)
    return out.reshape(b, s, HIDDEN)


# speedup vs baseline: 10.9195x; 1.1391x over previous
import functools

import jax
import jax.numpy as jnp
from jax import lax
from jax.experimental import pallas as pl
from jax.experimental.pallas import tpu as pltpu

HIDDEN = 1024
ROWS = 4 * 4096
BLK = 512
NBLK = ROWS // BLK


def _tc_body(ids_ref, table_ref, out_ref):
    sel = ids_ref[...] == 1
    t0 = table_ref[0][None, :]
    t1 = table_ref[1][None, :]
    out_ref[...] = jnp.where(sel, t1, t0)


@jax.jit
def kernel(token_type_ids, table):
    b, s = token_type_ids.shape
    ids = token_type_ids.astype(jnp.int32).reshape(ROWS, 1)
    out = pl.pallas_call(
        _tc_body,
        grid=(NBLK,),
        in_specs=[
            pl.BlockSpec((BLK, 1), lambda i: (i, 0)),
            pl.BlockSpec((2, HIDDEN), lambda i: (0, 0)),
        ],
        out_specs=pl.BlockSpec((BLK, HIDDEN), lambda i: (i, 0)),
        out_shape=jax.ShapeDtypeStruct((ROWS, HIDDEN), jnp.float32),
        compiler_params=pltpu.CompilerParams(
            dimension_semantics=("arbitrary",),
        ),
    )(ids, table)
    return out.reshape(b, s, HIDDEN)
